# Initial kernel scaffold; baseline (speedup 1.0000x reference)
import functools

import jax
import jax.numpy as jnp
from jax import lax
from jax.experimental import pallas as pl
from jax.experimental.pallas import tpu as pltpu
from jax.experimental.pallas import tpu_sc as plsc

VOCAB = 1000
EMB = 64
HID = 128
B = 1024
T = 50
BT = B * T

NC = 2
NS = 16
NW = NC * NS
PER_W = BT // NW
CH = 64
NCH = PER_W // CH

_sc_mesh = plsc.VectorSubcoreMesh(
    core_axis_name="c", subcore_axis_name="s", num_cores=NC, num_subcores=NS
)


@functools.partial(
    pl.kernel,
    out_type=jax.ShapeDtypeStruct((BT, EMB), jnp.float32),
    mesh=_sc_mesh,
    scratch_types=[
        pltpu.VMEM((NCH, CH), jnp.int32),
        pltpu.VMEM((PER_W, EMB), jnp.float32),
        pltpu.SemaphoreType.DMA,
    ],
)
def _sc_gather(idx_hbm, emb_hbm, out_hbm, idx_v, rows_v, sem):
    wid = lax.axis_index("s") * NC + lax.axis_index("c")
    pltpu.sync_copy(idx_hbm.at[wid], idx_v)

    def fire(j, carry):
        pltpu.async_copy(emb_hbm.at[idx_v.at[j]], rows_v.at[pl.ds(j * CH, CH)], sem)
        return carry

    lax.fori_loop(0, NCH, fire, 0)

    def drain(j, carry):
        pltpu.make_async_copy(
            emb_hbm.at[idx_v.at[j]], rows_v.at[pl.ds(j * CH, CH)], sem
        ).wait()
        return carry

    lax.fori_loop(0, NCH, drain, 0)
    pltpu.sync_copy(rows_v, out_hbm.at[pl.ds(wid * PER_W, PER_W)])


BBLK = 256
TCH = 10
NB = B // BBLK
NT = T // TCH


def _lstm_body(ex_ref, wih_ref, whh_ref, bih_ref, bhh_ref, wlin_ref, blin_ref,
               out_ref, hn_ref, cn_ref, h_ref, c_ref):
    tj = pl.program_id(1)

    @pl.when(tj == 0)
    def _():
        h_ref[...] = jnp.zeros_like(h_ref)
        c_ref[...] = jnp.zeros_like(c_ref)

    b = bih_ref[...] + bhh_ref[...]
    wih = wih_ref[...]
    whh = whh_ref[...]
    wlin = wlin_ref[...]
    blin = blin_ref[...]
    h = h_ref[...]
    c = c_ref[...]
    for t in range(TCH):
        xt = ex_ref[:, t, :]
        gates = (jnp.dot(xt, wih, preferred_element_type=jnp.float32)
                 + jnp.dot(h, whh, preferred_element_type=jnp.float32) + b)
        i = jax.nn.sigmoid(gates[:, :HID])
        f = jax.nn.sigmoid(gates[:, HID:2 * HID])
        g = jnp.tanh(gates[:, 2 * HID:3 * HID])
        o = jax.nn.sigmoid(gates[:, 3 * HID:])
        c = f * c + i * g
        h = o * jnp.tanh(c)
        out_ref[:, t, :] = (
            jnp.dot(h, wlin, preferred_element_type=jnp.float32) + blin
        )
    h_ref[...] = h
    c_ref[...] = c
    hn_ref[...] = h
    cn_ref[...] = c


def _lstm_call(embeds3, wihT, whhT, bih2, bhh2, wlinT, blin2):
    return pl.pallas_call(
        _lstm_body,
        grid=(NB, NT),
        in_specs=[
            pl.BlockSpec((BBLK, TCH, EMB), lambda i, j: (i, j, 0)),
            pl.BlockSpec((EMB, 4 * HID), lambda i, j: (0, 0)),
            pl.BlockSpec((HID, 4 * HID), lambda i, j: (0, 0)),
            pl.BlockSpec((1, 4 * HID), lambda i, j: (0, 0)),
            pl.BlockSpec((1, 4 * HID), lambda i, j: (0, 0)),
            pl.BlockSpec((HID, VOCAB), lambda i, j: (0, 0)),
            pl.BlockSpec((1, VOCAB), lambda i, j: (0, 0)),
        ],
        out_specs=[
            pl.BlockSpec((BBLK, TCH, VOCAB), lambda i, j: (i, j, 0)),
            pl.BlockSpec((BBLK, HID), lambda i, j: (i, 0)),
            pl.BlockSpec((BBLK, HID), lambda i, j: (i, 0)),
        ],
        out_shape=[
            jax.ShapeDtypeStruct((B, T, VOCAB), jnp.float32),
            jax.ShapeDtypeStruct((B, HID), jnp.float32),
            jax.ShapeDtypeStruct((B, HID), jnp.float32),
        ],
        scratch_shapes=[
            pltpu.VMEM((BBLK, HID), jnp.float32),
            pltpu.VMEM((BBLK, HID), jnp.float32),
        ],
        compiler_params=pltpu.CompilerParams(
            dimension_semantics=("arbitrary", "arbitrary"),
        ),
    )(embeds3, wihT, whhT, bih2, bhh2, wlinT, blin2)


def kernel(x, emb, w_ih, w_hh, b_ih, b_hh, w_lin, b_lin):
    idx = x.astype(jnp.int32).reshape(NW, NCH, CH)
    embeds = _sc_gather(idx, emb)
    embeds3 = embeds.reshape(B, T, EMB)
    logits3, hn, cn = _lstm_call(
        embeds3,
        w_ih.T,
        w_hh.T,
        b_ih.reshape(1, 4 * HID),
        b_hh.reshape(1, 4 * HID),
        w_lin.T,
        b_lin.reshape(1, VOCAB),
    )
    return (logits3.reshape(BT, VOCAB), hn[None], cn[None])


# speedup vs baseline: 1.7524x; 1.7524x over previous
import functools

import jax
import jax.numpy as jnp
from jax import lax
from jax.experimental import pallas as pl
from jax.experimental.pallas import tpu as pltpu
from jax.experimental.pallas import tpu_sc as plsc

VOCAB = 1000
EMB = 64
HID = 128
B = 1024
T = 50
BT = B * T

NC = 2
NS = 16
NW = NC * NS
EMB_P = 128
PER_W = BT // NW
CH = 80
NPASS = 2
HALF = PER_W // NPASS
NCH = HALF // CH


@functools.lru_cache(maxsize=1)
def _make_sc_gather():
    mesh = plsc.VectorSubcoreMesh(
        core_axis_name="c", subcore_axis_name="s", num_cores=NC, num_subcores=NS
    )

    @functools.partial(
        pl.kernel,
        out_type=jax.ShapeDtypeStruct((BT, EMB_P), jnp.float32),
        mesh=mesh,
        scratch_types=[
            pltpu.VMEM((PER_W,), jnp.int32),
            pltpu.VMEM((HALF, EMB_P), jnp.float32),
            pltpu.SemaphoreType.DMA,
        ],
    )
    def _sc_gather(idx_hbm, emb_hbm, out_hbm, idx_v, rows_v, sem):
        wid = lax.axis_index("s") * NC + lax.axis_index("c")
        base = wid * PER_W
        pltpu.sync_copy(idx_hbm.at[pl.ds(base, PER_W)], idx_v)

        for p in range(NPASS):
            def fire(j, carry, p=p):
                pltpu.async_copy(
                    emb_hbm.at[idx_v.at[pl.ds(p * HALF + j * CH, CH)]],
                    rows_v.at[pl.ds(j * CH, CH)],
                    sem,
                )
                return carry

            lax.fori_loop(0, NCH, fire, 0)

            def drain(j, carry, p=p):
                pltpu.make_async_copy(
                    emb_hbm.at[idx_v.at[pl.ds(p * HALF + j * CH, CH)]],
                    rows_v.at[pl.ds(j * CH, CH)],
                    sem,
                ).wait()
                return carry

            lax.fori_loop(0, NCH, drain, 0)
            pltpu.sync_copy(rows_v, out_hbm.at[pl.ds(base + p * HALF, HALF)])

    return _sc_gather


def _gather_embeds(idx, emb_padded):
    return _make_sc_gather()(idx, emb_padded)


BBLK = 256
TCH = 8
NB = B // BBLK
NT = (T + TCH - 1) // TCH


def _lstm_body(ex_ref, wih_ref, whh_ref, bih_ref, bhh_ref, wlin_ref, blin_ref,
               out_ref, hn_ref, cn_ref, h_ref, c_ref):
    tj = pl.program_id(1)

    @pl.when(tj == 0)
    def _():
        h_ref[...] = jnp.zeros_like(h_ref)
        c_ref[...] = jnp.zeros_like(c_ref)

    b = bih_ref[...] + bhh_ref[...]
    wih = wih_ref[...]
    whh = whh_ref[...]
    wlin = wlin_ref[...]
    blin = blin_ref[...]
    h = h_ref[...]
    c = c_ref[...]
    for t in range(TCH):
        valid = tj * TCH + t < T
        xt = ex_ref[:, t, :]
        gates = (jnp.dot(xt, wih, preferred_element_type=jnp.float32)
                 + jnp.dot(h, whh, preferred_element_type=jnp.float32) + b)
        i = jax.nn.sigmoid(gates[:, :HID])
        f = jax.nn.sigmoid(gates[:, HID:2 * HID])
        g = jnp.tanh(gates[:, 2 * HID:3 * HID])
        o = jax.nn.sigmoid(gates[:, 3 * HID:])
        c = jnp.where(valid, f * c + i * g, c)
        h = jnp.where(valid, o * jnp.tanh(c), h)
        out_ref[:, t, :] = (
            jnp.dot(h, wlin, preferred_element_type=jnp.float32) + blin
        )
    h_ref[...] = h
    c_ref[...] = c
    hn_ref[...] = h
    cn_ref[...] = c


def _lstm_call(embeds3, wihT, whhT, bih2, bhh2, wlinT, blin2):
    return pl.pallas_call(
        _lstm_body,
        grid=(NB, NT),
        in_specs=[
            pl.BlockSpec((BBLK, TCH, EMB_P), lambda i, j: (i, j, 0)),
            pl.BlockSpec((EMB_P, 4 * HID), lambda i, j: (0, 0)),
            pl.BlockSpec((HID, 4 * HID), lambda i, j: (0, 0)),
            pl.BlockSpec((1, 4 * HID), lambda i, j: (0, 0)),
            pl.BlockSpec((1, 4 * HID), lambda i, j: (0, 0)),
            pl.BlockSpec((HID, VOCAB), lambda i, j: (0, 0)),
            pl.BlockSpec((1, VOCAB), lambda i, j: (0, 0)),
        ],
        out_specs=[
            pl.BlockSpec((BBLK, TCH, VOCAB), lambda i, j: (i, j, 0)),
            pl.BlockSpec((BBLK, HID), lambda i, j: (i, 0)),
            pl.BlockSpec((BBLK, HID), lambda i, j: (i, 0)),
        ],
        out_shape=[
            jax.ShapeDtypeStruct((B, T, VOCAB), jnp.float32),
            jax.ShapeDtypeStruct((B, HID), jnp.float32),
            jax.ShapeDtypeStruct((B, HID), jnp.float32),
        ],
        scratch_shapes=[
            pltpu.VMEM((BBLK, HID), jnp.float32),
            pltpu.VMEM((BBLK, HID), jnp.float32),
        ],
        compiler_params=pltpu.CompilerParams(
            dimension_semantics=("arbitrary", "arbitrary"),
        ),
    )(embeds3, wihT, whhT, bih2, bhh2, wlinT, blin2)


def kernel(x, emb, w_ih, w_hh, b_ih, b_hh, w_lin, b_lin):
    idx = x.astype(jnp.int32).reshape(BT)
    emb_padded = jnp.pad(emb, ((0, 0), (0, EMB_P - EMB)))
    embeds = _gather_embeds(idx, emb_padded)
    embeds3 = embeds.reshape(B, T, EMB_P)
    wihT = jnp.pad(w_ih.T, ((0, EMB_P - EMB), (0, 0)))
    logits3, hn, cn = _lstm_call(
        embeds3,
        wihT,
        w_hh.T,
        b_ih.reshape(1, 4 * HID),
        b_hh.reshape(1, 4 * HID),
        w_lin.T,
        b_lin.reshape(1, VOCAB),
    )
    return (logits3.reshape(BT, VOCAB), hn[None], cn[None])
